# Initial kernel scaffold; baseline (speedup 1.0000x reference)
import jax
import jax.numpy as jnp
from jax.experimental import pallas as pl


def kernel(h, adj, W1, a1_src, a1_dst, W2, a2_src, a2_dst):
    raise NotImplementedError("write your pallas kernel here")


# speedup vs baseline: 2.6759x; 2.6759x over previous
import jax
import jax.numpy as jnp
from jax.experimental import pallas as pl

_INTERPRET = False


def _mm_kernel(x_ref, w_ref, o_ref):
    o_ref[...] = jnp.dot(x_ref[...], w_ref[...],
                         preferred_element_type=jnp.float32)


def _l1_kernel(adj_ref, xf_ref, xb_ref, ed_ref, w2_ref, o_ref):
    mask = adj_ref[...] > 0.0
    outs = []
    for hd in range(2):
        x_h = xf_ref[:, hd * 32:(hd + 1) * 32]
        e = xb_ref[:, 64 + hd:65 + hd] + ed_ref[hd:hd + 1, :]
        e = jnp.where(e >= 0.0, e, 0.2 * e)
        p = jnp.where(mask, jnp.exp(e), 0.0)
        s = jnp.sum(p, axis=1, keepdims=True)
        outs.append(jnp.dot(p, x_h, preferred_element_type=jnp.float32) / s)
    h1 = jnp.concatenate(outs, axis=1)
    h1 = jnp.where(h1 > 0.0, h1, jnp.exp(jnp.minimum(h1, 0.0)) - 1.0)
    o_ref[...] = jnp.dot(h1, w2_ref[...],
                         preferred_element_type=jnp.float32)


def _l2_kernel(adj_ref, xf_ref, xb_ref, ed_ref, o_ref):
    mask = adj_ref[...] > 0.0
    x = xf_ref[:, 0:64]
    e = xb_ref[:, 64:65] + ed_ref[0:1, :]
    e = jnp.where(e >= 0.0, e, 0.2 * e)
    p = jnp.where(mask, jnp.exp(e), 0.0)
    s = jnp.sum(p, axis=1, keepdims=True)
    o_ref[...] = jnp.dot(p, x, preferred_element_type=jnp.float32) / s


def kernel(h, adj, W1, a1_src, a1_dst, W2, a2_src, a2_dst):
    n, f_in = h.shape
    hid = a1_src.shape[1]
    heads = a1_src.shape[0]
    out_dim = W2.shape[1]
    R = 200
    grid = n // R

    w1s = jnp.stack([W1[:, k * hid:(k + 1) * hid] @ a1_src[k]
                     for k in range(heads)], axis=1)
    w1d = jnp.stack([W1[:, k * hid:(k + 1) * hid] @ a1_dst[k]
                     for k in range(heads)], axis=1)
    W1cat = jnp.concatenate(
        [W1, w1s, w1d, jnp.zeros((f_in, 128 - heads * hid - 4), jnp.float32)],
        axis=1)
    W2cat = jnp.concatenate(
        [W2, W2 @ a2_src[0][:, None], W2 @ a2_dst[0][:, None],
         jnp.zeros((heads * hid, 128 - out_dim - 2), jnp.float32)],
        axis=1)

    RM = 400
    X1e = pl.pallas_call(
        _mm_kernel,
        grid=(n // RM,),
        in_specs=[pl.BlockSpec((RM, f_in), lambda i: (i, 0)),
                  pl.BlockSpec((f_in, 128), lambda i: (0, 0))],
        out_specs=pl.BlockSpec((RM, 128), lambda i: (i, 0)),
        out_shape=jax.ShapeDtypeStruct((n, 128), jnp.float32),
        interpret=_INTERPRET,
    )(h, W1cat)

    ed1 = jnp.zeros((8, n), jnp.float32).at[0:2, :].set(X1e[:, 66:68].T)

    X2e = pl.pallas_call(
        _l1_kernel,
        grid=(grid,),
        in_specs=[pl.BlockSpec((R, n), lambda i: (i, 0)),
                  pl.BlockSpec((n, 128), lambda i: (0, 0)),
                  pl.BlockSpec((R, 128), lambda i: (i, 0)),
                  pl.BlockSpec((8, n), lambda i: (0, 0)),
                  pl.BlockSpec((heads * hid, 128), lambda i: (0, 0))],
        out_specs=pl.BlockSpec((R, 128), lambda i: (i, 0)),
        out_shape=jax.ShapeDtypeStruct((n, 128), jnp.float32),
        interpret=_INTERPRET,
    )(adj, X1e, X1e, ed1, W2cat)

    ed2 = jnp.zeros((8, n), jnp.float32).at[0:1, :].set(X2e[:, 65:66].T)

    out = pl.pallas_call(
        _l2_kernel,
        grid=(grid,),
        in_specs=[pl.BlockSpec((R, n), lambda i: (i, 0)),
                  pl.BlockSpec((n, 128), lambda i: (0, 0)),
                  pl.BlockSpec((R, 128), lambda i: (i, 0)),
                  pl.BlockSpec((8, n), lambda i: (0, 0))],
        out_specs=pl.BlockSpec((R, out_dim), lambda i: (i, 0)),
        out_shape=jax.ShapeDtypeStruct((n, out_dim), jnp.float32),
        interpret=_INTERPRET,
    )(adj, X2e, X2e, ed2)

    return out
